# baseline (speedup 1.0000x reference)
import functools

import jax
import jax.numpy as jnp
import numpy as np
from jax import lax
from jax.experimental import pallas as pl
from jax.experimental.pallas import tpu as pltpu
from jax.experimental.pallas import tpu_sc as plsc

DIM = 64
MAX_POS = 8192
THETA = 10000.0
SEQ = 8192

_inv_freq = 1.0 / (THETA ** (np.arange(0, DIM, 2, dtype=np.float64) / DIM))
_emb = np.concatenate([np.outer(np.arange(MAX_POS), _inv_freq)] * 2, axis=1)
_TAB = np.concatenate([np.cos(_emb), np.sin(_emb)], axis=1).astype(np.float32)

_NC, _NS = 2, 16
_NW = _NC * _NS
_CHUNK = SEQ // _NW


@functools.partial(
    pl.kernel,
    mesh=plsc.VectorSubcoreMesh(core_axis_name="c", subcore_axis_name="s"),
    out_type=(
        jax.ShapeDtypeStruct((SEQ, DIM), jnp.float32),
        jax.ShapeDtypeStruct((SEQ, DIM), jnp.float32),
    ),
    scratch_types=[
        pltpu.VMEM((_CHUNK,), jnp.int32),
        pltpu.VMEM((_CHUNK, 2 * DIM), jnp.float32),
        pltpu.SemaphoreType.DMA,
    ],
    compiler_params=pltpu.CompilerParams(use_tc_tiling_on_sc=False),
)
def _rope_gather(tab_hbm, idx_hbm, cos_out, sin_out, idx_v, rows_v, sem):
    wid = lax.axis_index("s") * _NC + lax.axis_index("c")
    base = wid * _CHUNK
    pltpu.sync_copy(idx_hbm.at[pl.ds(base, _CHUNK)], idx_v)
    pltpu.async_copy(tab_hbm.at[idx_v], rows_v, sem).wait()
    pltpu.sync_copy(rows_v.at[:, pl.ds(0, DIM)], cos_out.at[pl.ds(base, _CHUNK)])
    pltpu.sync_copy(rows_v.at[:, pl.ds(DIM, DIM)], sin_out.at[pl.ds(base, _CHUNK)])


def kernel(x, position_ids):
    tab = jnp.asarray(_TAB)
    idx = position_ids.reshape(SEQ).astype(jnp.int32)
    cos, sin = _rope_gather(tab, idx)
    cos = cos.reshape(1, 1, SEQ, DIM).astype(x.dtype)
    sin = sin.reshape(1, 1, SEQ, DIM).astype(x.dtype)
    return cos, sin
